# baseline (speedup 1.0000x reference)
import jax
import jax.numpy as jnp
from jax.experimental import pallas as pl
from jax.experimental.pallas import tpu as pltpu

TOKENS = 32768
D_MODEL = 1024
NUM_EXPERTS = 64
THRESHOLD = 0.5
TILE = 1024
NT = TOKENS // TILE
NBUF = 4


NCHUNK = 4
CHUNK = TILE // NCHUNK


def _start_copy(x_hbm, xbuf, sems, tile_idx, slot):
    for c in range(NCHUNK):
        pltpu.make_async_copy(
            x_hbm.at[pl.ds(tile_idx * TILE + c * CHUNK, CHUNK), :],
            xbuf.at[slot, pl.ds(c * CHUNK, CHUNK)],
            sems.at[slot, c],
        ).start()


def _gate_kernel(x_hbm, wt_ref, b_ref, gated_ref, mask_ref, xbuf, sems):
    i = pl.program_id(0)
    slot = jax.lax.rem(i, NBUF)

    @pl.when(i == 0)
    def _prologue():
        for k in range(min(NBUF, NT)):
            _start_copy(x_hbm, xbuf, sems, k, k)

    for c in range(NCHUNK):
        pltpu.make_async_copy(
            x_hbm.at[pl.ds(i * TILE + c * CHUNK, CHUNK), :],
            xbuf.at[slot, pl.ds(c * CHUNK, CHUNK)],
            sems.at[slot, c],
        ).wait()

    logits = jnp.dot(xbuf[slot], wt_ref[...], preferred_element_type=jnp.float32)
    logits = logits + b_ref[...]
    probs = jax.nn.sigmoid(logits)
    mask = (probs > THRESHOLD).astype(jnp.float32)
    gated_ref[...] = probs * mask
    mask_ref[...] = mask

    @pl.when(i + NBUF < NT)
    def _prefetch():
        _start_copy(x_hbm, xbuf, sems, i + NBUF, slot)


def kernel(x, W, b):
    wt = W.T
    b2 = b.reshape(1, NUM_EXPERTS)
    out_shape = jax.ShapeDtypeStruct((TOKENS, NUM_EXPERTS), jnp.float32)
    gated, mask = pl.pallas_call(
        _gate_kernel,
        grid=(NT,),
        in_specs=[
            pl.BlockSpec(memory_space=pl.ANY),
            pl.BlockSpec((D_MODEL, NUM_EXPERTS), lambda i: (0, 0)),
            pl.BlockSpec((1, NUM_EXPERTS), lambda i: (0, 0)),
        ],
        out_specs=[
            pl.BlockSpec((TILE, NUM_EXPERTS), lambda i: (i, 0)),
            pl.BlockSpec((TILE, NUM_EXPERTS), lambda i: (i, 0)),
        ],
        out_shape=[out_shape, out_shape],
        scratch_shapes=[
            pltpu.VMEM((NBUF, TILE, D_MODEL), jnp.float32),
            pltpu.SemaphoreType.DMA((NBUF, NCHUNK)),
        ],
        compiler_params=pltpu.CompilerParams(
            dimension_semantics=("arbitrary",),
        ),
    )(x, wt, b2)
    return gated, mask


# speedup vs baseline: 1.0032x; 1.0032x over previous
import jax
import jax.numpy as jnp
from jax.experimental import pallas as pl
from jax.experimental.pallas import tpu as pltpu

TOKENS = 32768
D_MODEL = 1024
NUM_EXPERTS = 64
THRESHOLD = 0.5
TILE = 2048


def _gate_kernel(x_ref, wt_ref, b_ref, gated_ref, mask_ref):
    logits = jnp.dot(x_ref[...], wt_ref[...], preferred_element_type=jnp.float32)
    logits = logits + b_ref[...]
    probs = jax.nn.sigmoid(logits)
    mask = (probs > THRESHOLD).astype(jnp.float32)
    gated_ref[...] = probs * mask
    mask_ref[...] = mask


def kernel(x, W, b):
    wt = W.T
    b2 = b.reshape(1, NUM_EXPERTS)
    grid = (TOKENS // TILE,)
    out_shape = jax.ShapeDtypeStruct((TOKENS, NUM_EXPERTS), jnp.float32)
    gated, mask = pl.pallas_call(
        _gate_kernel,
        grid=grid,
        in_specs=[
            pl.BlockSpec((TILE, D_MODEL), lambda i: (i, 0)),
            pl.BlockSpec((D_MODEL, NUM_EXPERTS), lambda i: (0, 0)),
            pl.BlockSpec((1, NUM_EXPERTS), lambda i: (0, 0)),
        ],
        out_specs=[
            pl.BlockSpec((TILE, NUM_EXPERTS), lambda i: (i, 0)),
            pl.BlockSpec((TILE, NUM_EXPERTS), lambda i: (i, 0)),
        ],
        out_shape=[out_shape, out_shape],
        compiler_params=pltpu.CompilerParams(
            dimension_semantics=("parallel",),
        ),
    )(x, wt, b2)
    return gated, mask
